# baseline (speedup 1.0000x reference)
import functools

import jax
import jax.numpy as jnp
from jax import lax
from jax.experimental import pallas as pl
from jax.experimental.pallas import tpu as pltpu
from jax.experimental.pallas import tpu_sc as plsc

_VOCAB = 30522
_HIDDEN = 768
_MAX_POS = 512
_BATCH = 10
_B = _BATCH * _MAX_POS
_NC, _NS = 2, 16
_NW = _NC * _NS
_HALF = _B // 2
_B_PER_W = _HALF // _NW

_TOK_BLK = 1024


_ROWS_W = _B // _NW
_CHUNK = _ROWS_W // 2


def _sc_gather(word_table, idx2d, start, ntok):
    mesh = plsc.VectorSubcoreMesh(core_axis_name="c", subcore_axis_name="s")
    rows_w = ntok // _NW

    @functools.partial(
        pl.kernel,
        mesh=mesh,
        out_type=jax.ShapeDtypeStruct((ntok, _HIDDEN), jnp.float32),
        scratch_types=[
            pltpu.VMEM((2 * _MAX_POS,), jnp.int32),
            pltpu.VMEM((rows_w, _HIDDEN), jnp.float32),
            pltpu.SemaphoreType.DMA,
        ],
    )
    def gather_kernel(table_hbm, idx_hbm, out_hbm, idx_v, rows_v, sem):
        wid = lax.axis_index("s") * _NC + lax.axis_index("c")
        base = wid * rows_w
        tok0 = start + base
        row = jnp.minimum(tok0 // _MAX_POS, _BATCH - 2)
        off = tok0 - row * _MAX_POS
        pltpu.sync_copy(idx_hbm.at[row], idx_v.at[pl.ds(0, _MAX_POS)])
        pltpu.sync_copy(idx_hbm.at[row + 1], idx_v.at[pl.ds(_MAX_POS, _MAX_POS)])
        pltpu.async_copy(table_hbm.at[idx_v.at[pl.ds(off, rows_w)]],
                         rows_v, sem).wait()
        pltpu.sync_copy(rows_v, out_hbm.at[pl.ds(base, rows_w)])

    return gather_kernel(word_table, idx2d)


def _tc_body(word_ref, pos_ref, seg_ref, type_ref, gam_ref, bet_ref, *rest):
    out_ref = rest[-1]
    pos = pos_ref[...]
    t0 = type_ref[0, :][None, :]
    t1 = type_ref[1, :][None, :]
    gam = gam_ref[0, :][None, :]
    bet = bet_ref[0, :][None, :]
    seg_all = seg_ref[0].astype(jnp.float32)
    for k in range(_TOK_BLK // _MAX_POS):
        x = word_ref[pl.ds(k * _MAX_POS, _MAX_POS), :] + pos
        seg_row = seg_all[:, k * _MAX_POS:(k + 1) * _MAX_POS]
        seg_sq = jnp.broadcast_to(seg_row, (128, _MAX_POS))
        segc = seg_sq.T[:, 0:1]
        x = x + (t0 + segc * (t1 - t0))
        mean = jnp.mean(x, axis=1, keepdims=True)
        xc = x - mean
        var = jnp.mean(xc * xc, axis=1, keepdims=True)
        y = xc * lax.rsqrt(var + 1e-5)
        out_ref[k] = y * gam + bet


def _tc_half(gathered, seg3, pos_table, type_table, gamma2, beta2,
             blk_offset, prev=None):
    nblk = gathered.shape[0] // _TOK_BLK
    in_specs = [
        pl.BlockSpec((_TOK_BLK, _HIDDEN), lambda i: (i, 0)),
        pl.BlockSpec((_MAX_POS, _HIDDEN), lambda i: (0, 0)),
        pl.BlockSpec((1, 1, _TOK_BLK), lambda i: (i, 0, 0)),
        pl.BlockSpec((2, _HIDDEN), lambda i: (0, 0)),
        pl.BlockSpec((1, _HIDDEN), lambda i: (0, 0)),
        pl.BlockSpec((1, _HIDDEN), lambda i: (0, 0)),
    ]
    args = [gathered, pos_table, seg3, type_table, gamma2, beta2]
    kwargs = {}
    if prev is not None:
        in_specs.append(pl.BlockSpec(memory_space=pl.ANY))
        args.append(prev)
        kwargs["input_output_aliases"] = {6: 0}
    return pl.pallas_call(
        _tc_body,
        grid=(nblk,),
        in_specs=in_specs,
        out_specs=pl.BlockSpec((_TOK_BLK // _MAX_POS, _MAX_POS, _HIDDEN),
                               lambda i: (i + blk_offset, 0, 0)),
        out_shape=jax.ShapeDtypeStruct((_BATCH, _MAX_POS, _HIDDEN),
                                       jnp.float32),
        **kwargs,
    )(*args)


_H1 = 2048
_H1_BLKS = _H1 // _TOK_BLK


def kernel(batch_idx, batch_seg_idx, word_table, pos_table, type_table,
           ln_gamma, ln_beta):
    idx2d = batch_idx.astype(jnp.int32)
    g1 = _sc_gather(word_table, idx2d, 0, _H1)
    g2 = _sc_gather(word_table, idx2d, _H1, _B - _H1)
    seg = batch_seg_idx.reshape(_B // _TOK_BLK, 1, _TOK_BLK).astype(jnp.int32)
    gamma2 = ln_gamma.reshape(1, _HIDDEN)
    beta2 = ln_beta.reshape(1, _HIDDEN)
    o1 = _tc_half(g1, seg[:_H1_BLKS], pos_table, type_table,
                  gamma2, beta2, 0)
    return _tc_half(g2, seg[_H1_BLKS:], pos_table, type_table,
                    gamma2, beta2, _H1_BLKS, prev=o1)


# speedup vs baseline: 1.0791x; 1.0791x over previous
import functools

import jax
import jax.numpy as jnp
from jax import lax
from jax.experimental import pallas as pl
from jax.experimental.pallas import tpu as pltpu
from jax.experimental.pallas import tpu_sc as plsc

_VOCAB = 30522
_HIDDEN = 768
_MAX_POS = 512
_BATCH = 10
_B = _BATCH * _MAX_POS
_NC, _NS = 2, 16
_NW = _NC * _NS
_HALF = _B // 2
_B_PER_W = _HALF // _NW

_TOK_BLK = 1024


_ROWS_W = _B // _NW
_CHUNK = _ROWS_W // 2


def _sc_gather(word_table, flat_idx):
    mesh = plsc.VectorSubcoreMesh(core_axis_name="c", subcore_axis_name="s")

    @functools.partial(
        pl.kernel,
        mesh=mesh,
        out_type=jax.ShapeDtypeStruct((_B, _HIDDEN), jnp.float32),
        scratch_types=[
            pltpu.VMEM((2 * _MAX_POS,), jnp.int32),
            pltpu.VMEM((_ROWS_W, _HIDDEN), jnp.float32),
            pltpu.SemaphoreType.DMA,
        ],
    )
    def gather_kernel(table_hbm, idx_hbm, out_hbm, idx_v, rows_v, sem):
        wid = lax.axis_index("s") * _NC + lax.axis_index("c")
        base = wid * _ROWS_W
        row = jnp.minimum(base // _MAX_POS, _BATCH - 2)
        off = base - row * _MAX_POS
        pltpu.sync_copy(idx_hbm.at[row], idx_v.at[pl.ds(0, _MAX_POS)])
        pltpu.sync_copy(idx_hbm.at[row + 1], idx_v.at[pl.ds(_MAX_POS, _MAX_POS)])
        pltpu.async_copy(table_hbm.at[idx_v.at[pl.ds(off, _ROWS_W)]],
                         rows_v, sem).wait()
        pltpu.sync_copy(rows_v, out_hbm.at[pl.ds(base, _ROWS_W)])

    return gather_kernel(word_table, flat_idx)


def _tc_body(word_ref, pos_ref, seg_ref, type_ref, gam_ref, bet_ref, *rest):
    out_ref = rest[-1]
    pos = pos_ref[...]
    t0 = type_ref[0, :][None, :]
    t1 = type_ref[1, :][None, :]
    gam = gam_ref[0, :][None, :]
    bet = bet_ref[0, :][None, :]
    seg_all = seg_ref[0].astype(jnp.float32)
    for k in range(_TOK_BLK // _MAX_POS):
        x = word_ref[pl.ds(k * _MAX_POS, _MAX_POS), :] + pos
        seg_row = seg_all[:, k * _MAX_POS:(k + 1) * _MAX_POS]
        seg_sq = jnp.broadcast_to(seg_row, (128, _MAX_POS))
        segc = seg_sq.T[:, 0:1]
        x = x + (t0 + segc * (t1 - t0))
        mean = jnp.mean(x, axis=1, keepdims=True)
        xc = x - mean
        var = jnp.mean(xc * xc, axis=1, keepdims=True)
        y = xc * lax.rsqrt(var + 1e-5)
        out_ref[k] = y * gam + bet


def _tc_finish(gathered, seg3, pos_table, type_table, gamma2, beta2):
    return pl.pallas_call(
        _tc_body,
        grid=(_B // _TOK_BLK,),
        in_specs=[
            pl.BlockSpec((_TOK_BLK, _HIDDEN), lambda i: (i, 0)),
            pl.BlockSpec((_MAX_POS, _HIDDEN), lambda i: (0, 0)),
            pl.BlockSpec((1, 1, _TOK_BLK), lambda i: (i, 0, 0)),
            pl.BlockSpec((2, _HIDDEN), lambda i: (0, 0)),
            pl.BlockSpec((1, _HIDDEN), lambda i: (0, 0)),
            pl.BlockSpec((1, _HIDDEN), lambda i: (0, 0)),
        ],
        out_specs=pl.BlockSpec((_TOK_BLK // _MAX_POS, _MAX_POS, _HIDDEN),
                               lambda i: (i, 0, 0)),
        out_shape=jax.ShapeDtypeStruct((_BATCH, _MAX_POS, _HIDDEN),
                                       jnp.float32),
    )(gathered, pos_table, seg3, type_table, gamma2, beta2)


def kernel(batch_idx, batch_seg_idx, word_table, pos_table, type_table,
           ln_gamma, ln_beta):
    gathered = _sc_gather(word_table, batch_idx.astype(jnp.int32))
    seg = batch_seg_idx.reshape(_B // _TOK_BLK, 1, _TOK_BLK).astype(jnp.int32)
    gamma2 = ln_gamma.reshape(1, _HIDDEN)
    beta2 = ln_beta.reshape(1, _HIDDEN)
    return _tc_finish(gathered, seg, pos_table, type_table, gamma2, beta2)


# speedup vs baseline: 1.0913x; 1.0113x over previous
import functools

import jax
import jax.numpy as jnp
from jax import lax
from jax.experimental import pallas as pl
from jax.experimental.pallas import tpu as pltpu
from jax.experimental.pallas import tpu_sc as plsc

_VOCAB = 30522
_HIDDEN = 768
_MAX_POS = 512
_BATCH = 10
_B = _BATCH * _MAX_POS
_NC, _NS = 2, 16
_NW = _NC * _NS
_HALF = _B // 2
_B_PER_W = _HALF // _NW

_TOK_BLK = 1024


_ROWS_W = _B // _NW
_CHUNK = _ROWS_W // 2


def _sc_gather(word_table, flat_idx):
    mesh = plsc.VectorSubcoreMesh(core_axis_name="c", subcore_axis_name="s")

    @functools.partial(
        pl.kernel,
        mesh=mesh,
        out_type=jax.ShapeDtypeStruct((_B, _HIDDEN), jnp.float32),
        scratch_types=[
            pltpu.VMEM((2 * _MAX_POS,), jnp.int32),
            pltpu.VMEM((_ROWS_W, _HIDDEN), jnp.float32),
            pltpu.SemaphoreType.DMA,
        ],
    )
    def gather_kernel(table_hbm, idx_hbm, out_hbm, idx_v, rows_v, sem):
        wid = lax.axis_index("s") * _NC + lax.axis_index("c")
        base = wid * _ROWS_W
        pltpu.sync_copy(idx_hbm.at[pl.ds(base, _ROWS_W)],
                        idx_v.at[pl.ds(0, _ROWS_W)])
        pltpu.async_copy(table_hbm.at[idx_v.at[pl.ds(0, _ROWS_W)]],
                         rows_v, sem).wait()
        pltpu.sync_copy(rows_v, out_hbm.at[pl.ds(base, _ROWS_W)])

    return gather_kernel(word_table, flat_idx)


def _tc_body(word_ref, pos_ref, seg_ref, type_ref, gam_ref, bet_ref, *rest):
    out_ref = rest[-1]
    pos = pos_ref[...]
    t0 = type_ref[0, :][None, :]
    t1 = type_ref[1, :][None, :]
    gam = gam_ref[0, :][None, :]
    bet = bet_ref[0, :][None, :]
    seg_all = seg_ref[0].astype(jnp.float32)
    for k in range(_TOK_BLK // _MAX_POS):
        x = word_ref[pl.ds(k * _MAX_POS, _MAX_POS), :] + pos
        seg_row = seg_all[:, k * _MAX_POS:(k + 1) * _MAX_POS]
        seg_sq = jnp.broadcast_to(seg_row, (128, _MAX_POS))
        segc = seg_sq.T[:, 0:1]
        x = x + (t0 + segc * (t1 - t0))
        mean = jnp.mean(x, axis=1, keepdims=True)
        xc = x - mean
        var = jnp.mean(xc * xc, axis=1, keepdims=True)
        y = xc * lax.rsqrt(var + 1e-5)
        out_ref[k] = y * gam + bet


def _tc_finish(gathered, seg3, pos_table, type_table, gamma2, beta2):
    return pl.pallas_call(
        _tc_body,
        grid=(_B // _TOK_BLK,),
        in_specs=[
            pl.BlockSpec((_TOK_BLK, _HIDDEN), lambda i: (i, 0)),
            pl.BlockSpec((_MAX_POS, _HIDDEN), lambda i: (0, 0)),
            pl.BlockSpec((1, 1, _TOK_BLK), lambda i: (i, 0, 0)),
            pl.BlockSpec((2, _HIDDEN), lambda i: (0, 0)),
            pl.BlockSpec((1, _HIDDEN), lambda i: (0, 0)),
            pl.BlockSpec((1, _HIDDEN), lambda i: (0, 0)),
        ],
        out_specs=pl.BlockSpec((_TOK_BLK // _MAX_POS, _MAX_POS, _HIDDEN),
                               lambda i: (i, 0, 0)),
        out_shape=jax.ShapeDtypeStruct((_BATCH, _MAX_POS, _HIDDEN),
                                       jnp.float32),
    )(gathered, pos_table, seg3, type_table, gamma2, beta2)


def kernel(batch_idx, batch_seg_idx, word_table, pos_table, type_table,
           ln_gamma, ln_beta):
    gathered = _sc_gather(word_table, batch_idx.reshape(-1).astype(jnp.int32))
    seg = batch_seg_idx.reshape(_B // _TOK_BLK, 1, _TOK_BLK).astype(jnp.int32)
    gamma2 = ln_gamma.reshape(1, _HIDDEN)
    beta2 = ln_beta.reshape(1, _HIDDEN)
    return _tc_finish(gathered, seg, pos_table, type_table, gamma2, beta2)


# speedup vs baseline: 1.1024x; 1.0101x over previous
import functools

import jax
import jax.numpy as jnp
from jax import lax
from jax.experimental import pallas as pl
from jax.experimental.pallas import tpu as pltpu
from jax.experimental.pallas import tpu_sc as plsc

_HIDDEN = 768
_MAX_POS = 512
_BATCH = 10
_B = _BATCH * _MAX_POS
_NC, _NS = 2, 16
_NW = _NC * _NS
_ROWS_W = _B // _NW
_TOK_BLK = 1024


def _sc_gather(word_table, flat_idx):
    mesh = plsc.VectorSubcoreMesh(core_axis_name="c", subcore_axis_name="s")

    @functools.partial(
        pl.kernel,
        mesh=mesh,
        out_type=jax.ShapeDtypeStruct((_B, _HIDDEN), jnp.float32),
        scratch_types=[
            pltpu.VMEM((_ROWS_W,), jnp.int32),
            pltpu.VMEM((_ROWS_W, _HIDDEN), jnp.float32),
            pltpu.SemaphoreType.DMA,
        ],
    )
    def gather_kernel(table_hbm, idx_hbm, out_hbm, idx_v, rows_v, sem):
        wid = lax.axis_index("s") * _NC + lax.axis_index("c")
        base = wid * _ROWS_W
        pltpu.sync_copy(idx_hbm.at[pl.ds(base, _ROWS_W)], idx_v)
        pltpu.async_copy(table_hbm.at[idx_v], rows_v, sem).wait()
        pltpu.sync_copy(rows_v, out_hbm.at[pl.ds(base, _ROWS_W)])

    return gather_kernel(word_table, flat_idx)


def _tc_body(word_ref, pos_ref, seg_ref, type_ref, gam_ref, bet_ref, out_ref):
    pos = pos_ref[...]
    t0 = type_ref[0, :][None, :]
    t1 = type_ref[1, :][None, :]
    gam = gam_ref[0, :][None, :]
    bet = bet_ref[0, :][None, :]
    seg_all = seg_ref[0].astype(jnp.float32)
    for k in range(_TOK_BLK // _MAX_POS):
        x = word_ref[pl.ds(k * _MAX_POS, _MAX_POS), :] + pos
        seg_row = seg_all[:, k * _MAX_POS:(k + 1) * _MAX_POS]
        seg_sq = jnp.broadcast_to(seg_row, (128, _MAX_POS))
        segc = seg_sq.T[:, 0:1]
        x = x + (t0 + segc * (t1 - t0))
        mean = jnp.mean(x, axis=1, keepdims=True)
        xc = x - mean
        var = jnp.mean(xc * xc, axis=1, keepdims=True)
        y = xc * lax.rsqrt(var + 1e-5)
        out_ref[k] = y * gam + bet


def _tc_finish(gathered, seg3, pos_table, type_table, gamma2, beta2):
    return pl.pallas_call(
        _tc_body,
        grid=(_B // _TOK_BLK,),
        in_specs=[
            pl.BlockSpec((_TOK_BLK, _HIDDEN), lambda i: (i, 0)),
            pl.BlockSpec((_MAX_POS, _HIDDEN), lambda i: (0, 0)),
            pl.BlockSpec((1, 1, _TOK_BLK), lambda i: (i, 0, 0)),
            pl.BlockSpec((2, _HIDDEN), lambda i: (0, 0)),
            pl.BlockSpec((1, _HIDDEN), lambda i: (0, 0)),
            pl.BlockSpec((1, _HIDDEN), lambda i: (0, 0)),
        ],
        out_specs=pl.BlockSpec((_TOK_BLK // _MAX_POS, _MAX_POS, _HIDDEN),
                               lambda i: (i, 0, 0)),
        out_shape=jax.ShapeDtypeStruct((_BATCH, _MAX_POS, _HIDDEN),
                                       jnp.float32),
    )(gathered, pos_table, seg3, type_table, gamma2, beta2)


def kernel(batch_idx, batch_seg_idx, word_table, pos_table, type_table,
           ln_gamma, ln_beta):
    gathered = _sc_gather(word_table, batch_idx.reshape(-1).astype(jnp.int32))
    seg = batch_seg_idx.reshape(_B // _TOK_BLK, 1, _TOK_BLK).astype(jnp.int32)
    gamma2 = ln_gamma.reshape(1, _HIDDEN)
    beta2 = ln_beta.reshape(1, _HIDDEN)
    return _tc_finish(gathered, seg, pos_table, type_table, gamma2, beta2)
